# baseline (speedup 1.0000x reference)
import jax
import jax.numpy as jnp
from jax.experimental import pallas as pl

D_MODEL = 768
N_HEADS = 12
D_HEAD = 64
N_CLUSTERS = 8
BS = 256
BQ = 256
NEG = -1e9


def _onehot_argmax(logits):
    idx = jax.lax.broadcasted_iota(jnp.int32, logits.shape, 1)
    mx = jnp.max(logits, axis=-1, keepdims=True)
    am = jnp.min(jnp.where(logits == mx, idx, logits.shape[-1]), axis=-1,
                 keepdims=True)
    return (idx == am).astype(jnp.float32)


def _proj_route_kernel(q_ref, k_ref, v_ref, wk_ref, wv_ref, wr_ref,
                       kp_ref, vp_ref, qoh_ref, koh_ref):
    qblk = q_ref[0]
    kblk = k_ref[0]
    vblk = v_ref[0]
    for h in range(N_HEADS):
        kp_ref[0, h] = jnp.dot(kblk, wk_ref[h],
                               preferred_element_type=jnp.float32)
        vp_ref[0, h] = jnp.dot(vblk, wv_ref[h],
                               preferred_element_type=jnp.float32)
    qoh_ref[0] = _onehot_argmax(
        jnp.dot(qblk, wr_ref[...], preferred_element_type=jnp.float32))
    koh_ref[0] = _onehot_argmax(
        jnp.dot(kblk, wr_ref[...], preferred_element_type=jnp.float32))


def _attn_kernel(q_ref, wq_ref, kp_ref, vp_ref, qoh_ref, koh_ref, wo_ref,
                 out_ref):
    h = pl.program_id(2)
    qh = jnp.dot(q_ref[0], wq_ref[0], preferred_element_type=jnp.float32)
    kh = kp_ref[0, 0]
    vh = vp_ref[0, 0]
    s = jax.lax.dot_general(qh, kh, (((1,), (1,)), ((), ())),
                            preferred_element_type=jnp.float32) * 0.125
    maskf = jax.lax.dot_general(qoh_ref[0], koh_ref[0],
                                (((1,), (1,)), ((), ())),
                                preferred_element_type=jnp.float32)
    s = jnp.where(maskf > 0.5, s, jnp.float32(NEG))
    m = jnp.max(s, axis=-1, keepdims=True)
    e = jnp.exp(s - m)
    p = e / jnp.sum(e, axis=-1, keepdims=True)
    ctx = jnp.dot(p, vh, preferred_element_type=jnp.float32)
    hasany = jnp.max(maskf, axis=-1, keepdims=True) > 0.5
    ctx = jnp.where(hasany, ctx, jnp.float32(0.0))
    contrib = jnp.dot(ctx, wo_ref[...], preferred_element_type=jnp.float32)

    @pl.when(h == 0)
    def _():
        out_ref[0] = contrib

    @pl.when(h != 0)
    def _():
        out_ref[0] += contrib


def kernel(q, k, v, Wq, Wk, Wv, Wo, Wr):
    B, Sq, D = q.shape
    Sk = k.shape[1]
    H, dh, M = N_HEADS, D_HEAD, N_CLUSTERS

    WqT = Wq.reshape(D, H, dh).transpose(1, 0, 2)
    WkT = Wk.reshape(D, H, dh).transpose(1, 0, 2)
    WvT = Wv.reshape(D, H, dh).transpose(1, 0, 2)

    nb = Sk // BS
    kp, vp, qoh, koh = pl.pallas_call(
        _proj_route_kernel,
        grid=(B, nb),
        in_specs=[
            pl.BlockSpec((1, BS, D), lambda b, i: (b, i, 0)),
            pl.BlockSpec((1, BS, D), lambda b, i: (b, i, 0)),
            pl.BlockSpec((1, BS, D), lambda b, i: (b, i, 0)),
            pl.BlockSpec((H, D, dh), lambda b, i: (0, 0, 0)),
            pl.BlockSpec((H, D, dh), lambda b, i: (0, 0, 0)),
            pl.BlockSpec((D, M), lambda b, i: (0, 0)),
        ],
        out_specs=[
            pl.BlockSpec((1, H, BS, dh), lambda b, i: (b, 0, i, 0)),
            pl.BlockSpec((1, H, BS, dh), lambda b, i: (b, 0, i, 0)),
            pl.BlockSpec((1, BS, M), lambda b, i: (b, i, 0)),
            pl.BlockSpec((1, BS, M), lambda b, i: (b, i, 0)),
        ],
        out_shape=[
            jax.ShapeDtypeStruct((B, H, Sk, dh), jnp.float32),
            jax.ShapeDtypeStruct((B, H, Sk, dh), jnp.float32),
            jax.ShapeDtypeStruct((B, Sq, M), jnp.float32),
            jax.ShapeDtypeStruct((B, Sk, M), jnp.float32),
        ],
    )(q, k, v, WkT, WvT, Wr)

    nq = Sq // BQ
    out = pl.pallas_call(
        _attn_kernel,
        grid=(B, nq, H),
        in_specs=[
            pl.BlockSpec((1, BQ, D), lambda b, i, h: (b, i, 0)),
            pl.BlockSpec((1, D, dh), lambda b, i, h: (h, 0, 0)),
            pl.BlockSpec((1, 1, Sk, dh), lambda b, i, h: (b, h, 0, 0)),
            pl.BlockSpec((1, 1, Sk, dh), lambda b, i, h: (b, h, 0, 0)),
            pl.BlockSpec((1, BQ, M), lambda b, i, h: (b, i, 0)),
            pl.BlockSpec((1, Sk, M), lambda b, i, h: (b, 0, 0)),
            pl.BlockSpec((dh, D), lambda b, i, h: (h, 0)),
        ],
        out_specs=pl.BlockSpec((1, BQ, D), lambda b, i, h: (b, i, 0)),
        out_shape=jax.ShapeDtypeStruct((B, Sq, D), jnp.float32),
    )(q, WqT, kp, vp, qoh, koh, Wo)
    return out


# speedup vs baseline: 1.0977x; 1.0977x over previous
import jax
import jax.numpy as jnp
from jax.experimental import pallas as pl

D_MODEL = 768
N_HEADS = 12
D_HEAD = 64
N_CLUSTERS = 8
BS = 256
BQ = 256
NEG = -1e9


def _onehot_argmax(logits):
    idx = jax.lax.broadcasted_iota(jnp.int32, logits.shape, 1)
    mx = jnp.max(logits, axis=-1, keepdims=True)
    am = jnp.min(jnp.where(logits == mx, idx, logits.shape[-1]), axis=-1,
                 keepdims=True)
    return (idx == am).astype(jnp.float32)


def _proj_route_kernel(q_ref, k_ref, v_ref, wk_ref, wv_ref, wr_ref,
                       kp_ref, vp_ref, qoh_ref, koh_ref):
    qblk = q_ref[0]
    kblk = k_ref[0]
    vblk = v_ref[0]
    kb16 = kblk.astype(jnp.bfloat16)
    vb16 = vblk.astype(jnp.bfloat16)
    for h in range(N_HEADS):
        kp_ref[0, h] = jnp.dot(kb16, wk_ref[h],
                               preferred_element_type=jnp.float32
                               ).astype(jnp.bfloat16)
        vp_ref[0, h] = jnp.dot(vb16, wv_ref[h],
                               preferred_element_type=jnp.float32
                               ).astype(jnp.bfloat16)
    qoh_ref[0] = _onehot_argmax(
        jnp.dot(qblk, wr_ref[...], preferred_element_type=jnp.float32))
    koh_ref[0] = _onehot_argmax(
        jnp.dot(kblk, wr_ref[...], preferred_element_type=jnp.float32))


def _attn_kernel(q_ref, wq_ref, kp_ref, vp_ref, qoh_ref, koh_ref, wo_ref,
                 out_ref):
    h = pl.program_id(2)
    qh = jnp.dot(q_ref[0].astype(jnp.bfloat16), wq_ref[0],
                 preferred_element_type=jnp.float32).astype(jnp.bfloat16)
    kh = kp_ref[0, 0]
    vh = vp_ref[0, 0]
    s = jax.lax.dot_general(qh, kh, (((1,), (1,)), ((), ())),
                            preferred_element_type=jnp.float32) * 0.125
    maskf = jax.lax.dot_general(qoh_ref[0], koh_ref[0],
                                (((1,), (1,)), ((), ())),
                                preferred_element_type=jnp.float32)
    s = jnp.where(maskf > 0.5, s, jnp.float32(NEG))
    m = jnp.max(s, axis=-1, keepdims=True)
    e = jnp.exp(s - m)
    p = (e / jnp.sum(e, axis=-1, keepdims=True)).astype(jnp.bfloat16)
    ctx = jnp.dot(p, vh, preferred_element_type=jnp.float32)
    hasany = jnp.max(maskf, axis=-1, keepdims=True) > 0.5
    ctx = jnp.where(hasany, ctx, jnp.float32(0.0))
    contrib = jnp.dot(ctx.astype(jnp.bfloat16), wo_ref[...],
                      preferred_element_type=jnp.float32)

    @pl.when(h == 0)
    def _():
        out_ref[0] = contrib

    @pl.when(h != 0)
    def _():
        out_ref[0] += contrib


def kernel(q, k, v, Wq, Wk, Wv, Wo, Wr):
    B, Sq, D = q.shape
    Sk = k.shape[1]
    H, dh, M = N_HEADS, D_HEAD, N_CLUSTERS

    WqT = Wq.reshape(D, H, dh).transpose(1, 0, 2).astype(jnp.bfloat16)
    WkT = Wk.reshape(D, H, dh).transpose(1, 0, 2).astype(jnp.bfloat16)
    WvT = Wv.reshape(D, H, dh).transpose(1, 0, 2).astype(jnp.bfloat16)
    Wo16 = Wo.astype(jnp.bfloat16)

    nb = Sk // BS
    kp, vp, qoh, koh = pl.pallas_call(
        _proj_route_kernel,
        grid=(B, nb),
        in_specs=[
            pl.BlockSpec((1, BS, D), lambda b, i: (b, i, 0)),
            pl.BlockSpec((1, BS, D), lambda b, i: (b, i, 0)),
            pl.BlockSpec((1, BS, D), lambda b, i: (b, i, 0)),
            pl.BlockSpec((H, D, dh), lambda b, i: (0, 0, 0)),
            pl.BlockSpec((H, D, dh), lambda b, i: (0, 0, 0)),
            pl.BlockSpec((D, M), lambda b, i: (0, 0)),
        ],
        out_specs=[
            pl.BlockSpec((1, H, BS, dh), lambda b, i: (b, 0, i, 0)),
            pl.BlockSpec((1, H, BS, dh), lambda b, i: (b, 0, i, 0)),
            pl.BlockSpec((1, BS, M), lambda b, i: (b, i, 0)),
            pl.BlockSpec((1, BS, M), lambda b, i: (b, i, 0)),
        ],
        out_shape=[
            jax.ShapeDtypeStruct((B, H, Sk, dh), jnp.bfloat16),
            jax.ShapeDtypeStruct((B, H, Sk, dh), jnp.bfloat16),
            jax.ShapeDtypeStruct((B, Sq, M), jnp.float32),
            jax.ShapeDtypeStruct((B, Sk, M), jnp.float32),
        ],
    )(q, k, v, WkT, WvT, Wr)

    nq = Sq // BQ
    out = pl.pallas_call(
        _attn_kernel,
        grid=(B, nq, H),
        in_specs=[
            pl.BlockSpec((1, BQ, D), lambda b, i, h: (b, i, 0)),
            pl.BlockSpec((1, D, dh), lambda b, i, h: (h, 0, 0)),
            pl.BlockSpec((1, 1, Sk, dh), lambda b, i, h: (b, h, 0, 0)),
            pl.BlockSpec((1, 1, Sk, dh), lambda b, i, h: (b, h, 0, 0)),
            pl.BlockSpec((1, BQ, M), lambda b, i, h: (b, i, 0)),
            pl.BlockSpec((1, Sk, M), lambda b, i, h: (b, 0, 0)),
            pl.BlockSpec((dh, D), lambda b, i, h: (h, 0)),
        ],
        out_specs=pl.BlockSpec((1, BQ, D), lambda b, i, h: (b, i, 0)),
        out_shape=jax.ShapeDtypeStruct((B, Sq, D), jnp.float32),
    )(q, WqT, kp, vp, qoh, koh, Wo16)
    return out


# speedup vs baseline: 1.6097x; 1.4664x over previous
import jax
import jax.numpy as jnp
from jax.experimental import pallas as pl

D_MODEL = 768
N_HEADS = 12
D_HEAD = 64
N_CLUSTERS = 8
BS = 256
BQ = 256
BIAS = 1024.0


def _onehot_argmax(logits):
    idx = jax.lax.broadcasted_iota(jnp.int32, logits.shape, 1)
    mx = jnp.max(logits, axis=-1, keepdims=True)
    am = jnp.min(jnp.where(logits == mx, idx, logits.shape[-1]), axis=-1,
                 keepdims=True)
    return (idx == am).astype(jnp.float32)


def _proj_route_kernel(q_ref, k_ref, v_ref, wk_ref, wv_ref, wr_ref,
                       kp_ref, vp_ref, qoh_ref, koh_ref):
    qblk = q_ref[0]
    kblk = k_ref[0]
    vblk = v_ref[0]
    kb16 = kblk.astype(jnp.bfloat16)
    vb16 = vblk.astype(jnp.bfloat16)
    for h in range(N_HEADS):
        kp_ref[0, h] = jnp.dot(kb16, wk_ref[h],
                               preferred_element_type=jnp.float32
                               ).astype(jnp.bfloat16)
        vp_ref[0, h] = jnp.dot(vb16, wv_ref[h],
                               preferred_element_type=jnp.float32
                               ).astype(jnp.bfloat16)
    qoh_ref[0] = _onehot_argmax(
        jnp.dot(qblk, wr_ref[...], preferred_element_type=jnp.float32)
    ).astype(jnp.bfloat16)
    koh_ref[0] = _onehot_argmax(
        jnp.dot(kblk, wr_ref[...], preferred_element_type=jnp.float32)
    ).astype(jnp.bfloat16)


def _attn_kernel(q_ref, wq_ref, kp_ref, vp_ref, qoh_ref, koh_ref, wo_ref,
                 out_ref):
    h = pl.program_id(2)
    qh = jnp.dot(q_ref[0].astype(jnp.bfloat16), wq_ref[0],
                 preferred_element_type=jnp.float32)
    qa = jnp.concatenate(
        [(qh * 0.125).astype(jnp.bfloat16), qoh_ref[0] * BIAS], axis=1)
    ka = jnp.concatenate([kp_ref[0, 0], koh_ref[0]], axis=1)
    s = jax.lax.dot_general(qa, ka, (((1,), (1,)), ((), ())),
                            preferred_element_type=jnp.float32)
    e = jnp.exp(s - BIAS)
    denom = jnp.sum(e, axis=-1, keepdims=True)
    ctx_un = jnp.dot(e.astype(jnp.bfloat16), vp_ref[0, 0],
                     preferred_element_type=jnp.float32)
    r = jnp.where(denom > 0.0, 1.0 / denom, jnp.float32(0.0))
    ctx = ctx_un * r
    contrib = jnp.dot(ctx.astype(jnp.bfloat16), wo_ref[...],
                      preferred_element_type=jnp.float32)

    @pl.when(h == 0)
    def _():
        out_ref[0] = contrib

    @pl.when(h != 0)
    def _():
        out_ref[0] += contrib


def kernel(q, k, v, Wq, Wk, Wv, Wo, Wr):
    B, Sq, D = q.shape
    Sk = k.shape[1]
    H, dh, M = N_HEADS, D_HEAD, N_CLUSTERS

    WqT = Wq.reshape(D, H, dh).transpose(1, 0, 2).astype(jnp.bfloat16)
    WkT = Wk.reshape(D, H, dh).transpose(1, 0, 2).astype(jnp.bfloat16)
    WvT = Wv.reshape(D, H, dh).transpose(1, 0, 2).astype(jnp.bfloat16)
    Wo16 = Wo.astype(jnp.bfloat16)

    nb = Sk // BS
    kp, vp, qoh, koh = pl.pallas_call(
        _proj_route_kernel,
        grid=(B, nb),
        in_specs=[
            pl.BlockSpec((1, BS, D), lambda b, i: (b, i, 0)),
            pl.BlockSpec((1, BS, D), lambda b, i: (b, i, 0)),
            pl.BlockSpec((1, BS, D), lambda b, i: (b, i, 0)),
            pl.BlockSpec((H, D, dh), lambda b, i: (0, 0, 0)),
            pl.BlockSpec((H, D, dh), lambda b, i: (0, 0, 0)),
            pl.BlockSpec((D, M), lambda b, i: (0, 0)),
        ],
        out_specs=[
            pl.BlockSpec((1, H, BS, dh), lambda b, i: (b, 0, i, 0)),
            pl.BlockSpec((1, H, BS, dh), lambda b, i: (b, 0, i, 0)),
            pl.BlockSpec((1, BS, M), lambda b, i: (b, i, 0)),
            pl.BlockSpec((1, BS, M), lambda b, i: (b, i, 0)),
        ],
        out_shape=[
            jax.ShapeDtypeStruct((B, H, Sk, dh), jnp.bfloat16),
            jax.ShapeDtypeStruct((B, H, Sk, dh), jnp.bfloat16),
            jax.ShapeDtypeStruct((B, Sq, M), jnp.bfloat16),
            jax.ShapeDtypeStruct((B, Sk, M), jnp.bfloat16),
        ],
    )(q, k, v, WkT, WvT, Wr)

    nq = Sq // BQ
    out = pl.pallas_call(
        _attn_kernel,
        grid=(B, nq, H),
        in_specs=[
            pl.BlockSpec((1, BQ, D), lambda b, i, h: (b, i, 0)),
            pl.BlockSpec((1, D, dh), lambda b, i, h: (h, 0, 0)),
            pl.BlockSpec((1, 1, Sk, dh), lambda b, i, h: (b, h, 0, 0)),
            pl.BlockSpec((1, 1, Sk, dh), lambda b, i, h: (b, h, 0, 0)),
            pl.BlockSpec((1, BQ, M), lambda b, i, h: (b, i, 0)),
            pl.BlockSpec((1, Sk, M), lambda b, i, h: (b, 0, 0)),
            pl.BlockSpec((dh, D), lambda b, i, h: (h, 0)),
        ],
        out_specs=pl.BlockSpec((1, BQ, D), lambda b, i, h: (b, i, 0)),
        out_shape=jax.ShapeDtypeStruct((B, Sq, D), jnp.float32),
    )(q, WqT, kp, vp, qoh, koh, Wo16)
    return out


# speedup vs baseline: 1.6441x; 1.0214x over previous
import jax
import jax.numpy as jnp
from jax.experimental import pallas as pl

D_MODEL = 768
N_HEADS = 12
D_HEAD = 64
N_CLUSTERS = 8
BS = 256
BQ = 512
BIAS = 1024.0


def _onehot_argmax(logits):
    idx = jax.lax.broadcasted_iota(jnp.int32, logits.shape, 1)
    mx = jnp.max(logits, axis=-1, keepdims=True)
    am = jnp.min(jnp.where(logits == mx, idx, logits.shape[-1]), axis=-1,
                 keepdims=True)
    return (idx == am).astype(jnp.float32)


def _proj_route_kernel(q_ref, k_ref, v_ref, wk_ref, wv_ref, wr_ref,
                       kp_ref, vp_ref, qoh_ref, koh_ref):
    qblk = q_ref[0]
    kblk = k_ref[0]
    vblk = v_ref[0]
    kb16 = kblk.astype(jnp.bfloat16)
    vb16 = vblk.astype(jnp.bfloat16)
    for h in range(N_HEADS):
        kp_ref[0, h] = jnp.dot(kb16, wk_ref[h],
                               preferred_element_type=jnp.float32
                               ).astype(jnp.bfloat16)
        vp_ref[0, h] = jnp.dot(vb16, wv_ref[h],
                               preferred_element_type=jnp.float32
                               ).astype(jnp.bfloat16)
    qoh_ref[0] = _onehot_argmax(
        jnp.dot(qblk, wr_ref[...], preferred_element_type=jnp.float32)
    ).astype(jnp.bfloat16)
    koh_ref[0] = _onehot_argmax(
        jnp.dot(kblk, wr_ref[...], preferred_element_type=jnp.float32)
    ).astype(jnp.bfloat16)


def _attn_kernel(q_ref, wq_ref, kp_ref, vp_ref, qoh_ref, koh_ref, wo_ref,
                 out_ref):
    h = pl.program_id(2)
    qh = jnp.dot(q_ref[0].astype(jnp.bfloat16), wq_ref[0],
                 preferred_element_type=jnp.float32)
    qa = jnp.concatenate(
        [(qh * 0.125).astype(jnp.bfloat16), qoh_ref[0] * BIAS], axis=1)
    kh = kp_ref[0, h]
    vh = vp_ref[0, h]
    ka = jnp.concatenate([kh, koh_ref[0]], axis=1)
    s = jax.lax.dot_general(qa, ka, (((1,), (1,)), ((), ())),
                            preferred_element_type=jnp.float32)
    e = jnp.exp(s - BIAS)
    denom = jnp.sum(e, axis=-1, keepdims=True)
    ctx_un = jnp.dot(e.astype(jnp.bfloat16), vh,
                     preferred_element_type=jnp.float32)
    r = jnp.where(denom > 0.0, 1.0 / denom, jnp.float32(0.0))
    ctx = ctx_un * r
    contrib = jnp.dot(ctx.astype(jnp.bfloat16), wo_ref[...],
                      preferred_element_type=jnp.float32)

    @pl.when(h == 0)
    def _():
        out_ref[0] = contrib

    @pl.when(h != 0)
    def _():
        out_ref[0] += contrib


def kernel(q, k, v, Wq, Wk, Wv, Wo, Wr):
    B, Sq, D = q.shape
    Sk = k.shape[1]
    H, dh, M = N_HEADS, D_HEAD, N_CLUSTERS

    WqT = Wq.reshape(D, H, dh).transpose(1, 0, 2).astype(jnp.bfloat16)
    WkT = Wk.reshape(D, H, dh).transpose(1, 0, 2).astype(jnp.bfloat16)
    WvT = Wv.reshape(D, H, dh).transpose(1, 0, 2).astype(jnp.bfloat16)
    Wo16 = Wo.astype(jnp.bfloat16)

    nb = Sk // BS
    kp, vp, qoh, koh = pl.pallas_call(
        _proj_route_kernel,
        grid=(B, nb),
        in_specs=[
            pl.BlockSpec((1, BS, D), lambda b, i: (b, i, 0)),
            pl.BlockSpec((1, BS, D), lambda b, i: (b, i, 0)),
            pl.BlockSpec((1, BS, D), lambda b, i: (b, i, 0)),
            pl.BlockSpec((H, D, dh), lambda b, i: (0, 0, 0)),
            pl.BlockSpec((H, D, dh), lambda b, i: (0, 0, 0)),
            pl.BlockSpec((D, M), lambda b, i: (0, 0)),
        ],
        out_specs=[
            pl.BlockSpec((1, H, BS, dh), lambda b, i: (b, 0, i, 0)),
            pl.BlockSpec((1, H, BS, dh), lambda b, i: (b, 0, i, 0)),
            pl.BlockSpec((1, BS, M), lambda b, i: (b, i, 0)),
            pl.BlockSpec((1, BS, M), lambda b, i: (b, i, 0)),
        ],
        out_shape=[
            jax.ShapeDtypeStruct((B, H, Sk, dh), jnp.bfloat16),
            jax.ShapeDtypeStruct((B, H, Sk, dh), jnp.bfloat16),
            jax.ShapeDtypeStruct((B, Sq, M), jnp.bfloat16),
            jax.ShapeDtypeStruct((B, Sk, M), jnp.bfloat16),
        ],
    )(q, k, v, WkT, WvT, Wr)

    nq = Sq // BQ
    out = pl.pallas_call(
        _attn_kernel,
        grid=(B, nq, H),
        in_specs=[
            pl.BlockSpec((1, BQ, D), lambda b, i, h: (b, i, 0)),
            pl.BlockSpec((1, D, dh), lambda b, i, h: (h, 0, 0)),
            pl.BlockSpec((1, H, Sk, dh), lambda b, i, h: (b, 0, 0, 0)),
            pl.BlockSpec((1, H, Sk, dh), lambda b, i, h: (b, 0, 0, 0)),
            pl.BlockSpec((1, BQ, M), lambda b, i, h: (b, i, 0)),
            pl.BlockSpec((1, Sk, M), lambda b, i, h: (b, 0, 0)),
            pl.BlockSpec((dh, D), lambda b, i, h: (h, 0)),
        ],
        out_specs=pl.BlockSpec((1, BQ, D), lambda b, i, h: (b, i, 0)),
        out_shape=jax.ShapeDtypeStruct((B, Sq, D), jnp.float32),
    )(q, WqT, kp, vp, qoh, koh, Wo16)
    return out
